# Initial kernel scaffold; baseline (speedup 1.0000x reference)
import jax
import jax.numpy as jnp
from jax.experimental import pallas as pl


def kernel(nodes, elemNodes, elems, elemConn, params):
    raise NotImplementedError("write your pallas kernel here")


# speedup vs baseline: 4.5700x; 4.5700x over previous
import functools

import jax
import jax.numpy as jnp
from jax import lax
from jax.experimental import pallas as pl
from jax.experimental.pallas import tpu as pltpu
from jax.experimental.pallas import tpu_sc as plsc

N = 50000
NE = 100000
K = 4
H = 64

NWORK = 32
CH = 128
PER_W = 12544
P = NWORK * PER_W
NCH_G = PER_W // CH

HALF = N // 2
TRASH = HALF
ACC_ROWS = HALF + 8
PER_T = P // 16
NCH_S = PER_T // CH
ROWS_T = 1562


def _gelu(x):
    return 0.5 * x * (1.0 + lax.erf(x * 0.7071067811865476))


def _ln(y, g, b):
    m = jnp.mean(y, axis=-1, keepdims=True)
    d = y - m
    v = jnp.mean(d * d, axis=-1, keepdims=True)
    return d * lax.rsqrt(v + 1e-5) * g + b


def _dot(a, b):
    return jnp.dot(a, b, preferred_element_type=jnp.float32)


def _mlp_rows(x, p, norm):
    R, nin = x.shape
    nout = p["l3"]["W"].shape[1]
    BR = 4000 if R % 4000 == 0 else 5000
    w = [p["l1"]["W"], p["l1"]["b"].reshape(1, -1),
         p["l2"]["W"], p["l2"]["b"].reshape(1, -1),
         p["l3"]["W"], p["l3"]["b"].reshape(1, -1)]
    if norm:
        w += [p["g"].reshape(1, -1), p["be"].reshape(1, -1)]

    def body(x_ref, *refs):
        o_ref = refs[-1]
        ws = [r[...] for r in refs[:-1]]
        h = _gelu(_dot(x_ref[...], ws[0]) + ws[1])
        h = _gelu(_dot(h, ws[2]) + ws[3])
        y = _dot(h, ws[4]) + ws[5]
        if norm:
            y = _ln(y, ws[6], ws[7])
        o_ref[...] = y

    full = lambda a: pl.BlockSpec(a.shape, lambda i: (0,) * a.ndim)
    return pl.pallas_call(
        body,
        grid=(R // BR,),
        in_specs=[pl.BlockSpec((BR, nin), lambda i: (i, 0))] + [full(a) for a in w],
        out_specs=pl.BlockSpec((BR, nout), lambda i: (i, 0)),
        out_shape=jax.ShapeDtypeStruct((R, nout), jnp.float32),
    )(x, *w)


def _layer_tc(lp, gathered, en_h, ex_h):
    BE = 1000
    BR = BE * K

    def split_w(p):
        W1 = p["l1"]["W"]
        return [W1[:H], W1[H:2 * H], W1[2 * H:], p["l1"]["b"].reshape(1, H),
                p["l2"]["W"], p["l2"]["b"].reshape(1, H),
                p["l3"]["W"], p["l3"]["b"].reshape(1, H),
                p["g"].reshape(1, H), p["be"].reshape(1, H)]

    ew = split_w(lp["elem"])
    nw = split_w(lp["node"])

    def bc4(t):
        return jnp.broadcast_to(t[:, None, :], (BE, K, H)).reshape(BR, H)

    def body(g_ref, en_ref, ex_ref, *refs):
        exo_ref, int_ref = refs[-2], refs[-1]
        ws = [r[...] for r in refs[:-2]]
        e = ws[:10]
        n = ws[10:]
        g = g_ref[...]
        en = en_ref[...]
        ex = ex_ref[...]
        gg_e = _dot(g, e[0]) + _dot(en, e[1])
        pre = gg_e + bc4(_dot(ex, e[2])) + e[3]
        h = _gelu(pre)
        h = _gelu(_dot(h, e[4]) + e[5])
        y = _ln(_dot(h, e[6]) + e[7], e[8], e[9])
        exn = ex + jnp.mean(y.reshape(BE, K, H), axis=1)
        exo_ref[...] = exn
        gg_n = _dot(g, n[0]) + _dot(en, n[1])
        pre2 = gg_n + bc4(_dot(exn, n[2])) + n[3]
        h2 = _gelu(pre2)
        h2 = _gelu(_dot(h2, n[4]) + n[5])
        int_ref[...] = _ln(_dot(h2, n[6]) + n[7], n[8], n[9])

    full = lambda a: pl.BlockSpec(a.shape, lambda i: (0,) * a.ndim)
    return pl.pallas_call(
        body,
        grid=(NE // BE,),
        in_specs=[pl.BlockSpec((BR, H), lambda i: (i, 0)),
                  pl.BlockSpec((BR, H), lambda i: (i, 0)),
                  pl.BlockSpec((BE, H), lambda i: (i, 0))]
                 + [full(a) for a in ew + nw],
        out_specs=[pl.BlockSpec((BE, H), lambda i: (i, 0)),
                   pl.BlockSpec((BR, H), lambda i: (i, 0))],
        out_shape=[jax.ShapeDtypeStruct((NE, H), jnp.float32),
                   jax.ShapeDtypeStruct((P, H), jnp.float32)],
    )(gathered, en_h, ex_h, *ew, *nw)


@functools.cache
def _sc_gather_kernel():
    mesh = plsc.VectorSubcoreMesh(core_axis_name="c", subcore_axis_name="s")

    @functools.partial(
        pl.kernel,
        out_type=jax.ShapeDtypeStruct((P, H), jnp.float32),
        mesh=mesh,
        scratch_types=[pltpu.VMEM((CH,), jnp.int32),
                       pltpu.VMEM((CH, H), jnp.float32),
                       pltpu.SemaphoreType.DMA],
        compiler_params=pltpu.CompilerParams(use_tc_tiling_on_sc=False),
    )
    def _sc_gather(tab, idx, out, idx_v, rows_v, sem):
        c = lax.axis_index("c")
        s = lax.axis_index("s")
        base = (s * 2 + c) * PER_W

        def body(i, carry):
            off = base + i * CH
            pltpu.sync_copy(idx.at[pl.ds(off, CH)], idx_v)
            pltpu.async_copy(tab.at[idx_v], rows_v, sem).wait()
            pltpu.sync_copy(rows_v, out.at[pl.ds(off, CH)])
            return carry

        lax.fori_loop(0, NCH_G, body, 0)

    return _sc_gather


@functools.cache
def _sc_scatter_kernel():
    mesh = plsc.VectorSubcoreMesh(core_axis_name="c", subcore_axis_name="s")

    @functools.partial(
        pl.kernel,
        out_type=jax.ShapeDtypeStruct((N, H), jnp.float32),
        mesh=mesh,
        scratch_types=[pltpu.VMEM((CH,), jnp.int32),
                       pltpu.VMEM((CH, H), jnp.float32),
                       pltpu.VMEM_SHARED((ACC_ROWS, H), jnp.float32),
                       pltpu.SemaphoreType.DMA],
        compiler_params=pltpu.CompilerParams(use_tc_tiling_on_sc=False),
    )
    def _sc_scatter(interim, scidx, nodes, out, idx_v, rows_v, acc, sem):
        c = lax.axis_index("c")
        s = lax.axis_index("s")
        pltpu.sync_copy(nodes.at[pl.ds(c * HALF + s * ROWS_T, ROWS_T)],
                        acc.at[pl.ds(s * ROWS_T, ROWS_T)])

        @pl.when(s == 15)
        def _():
            pltpu.sync_copy(
                nodes.at[pl.ds(c * HALF + 16 * ROWS_T, HALF - 16 * ROWS_T)],
                acc.at[pl.ds(16 * ROWS_T, HALF - 16 * ROWS_T)])

        plsc.subcore_barrier()

        def body(i, carry):
            off = s * PER_T + i * CH
            pltpu.sync_copy(scidx.at[c, pl.ds(off, CH)], idx_v)
            pltpu.sync_copy(interim.at[pl.ds(off, CH)], rows_v)
            pltpu.sync_copy(rows_v, acc.at[idx_v], add=True)
            return carry

        lax.fori_loop(0, NCH_S, body, 0)
        plsc.subcore_barrier()
        pltpu.sync_copy(acc.at[pl.ds(s * ROWS_T, ROWS_T)],
                        out.at[pl.ds(c * HALF + s * ROWS_T, ROWS_T)])

        @pl.when(s == 15)
        def _():
            pltpu.sync_copy(
                acc.at[pl.ds(16 * ROWS_T, HALF - 16 * ROWS_T)],
                out.at[pl.ds(c * HALF + 16 * ROWS_T, HALF - 16 * ROWS_T)])

    return _sc_scatter


def kernel(nodes, elemNodes, elems, elemConn, params):
    nodes2 = nodes[0]
    en2 = elemNodes.reshape(NE * K, -1)
    ex2 = elems[0]
    conn = elemConn[0]

    gat_idx = jnp.pad(conn.reshape(-1), (0, P - NE * K))
    idx_t = conn.T.reshape(-1)
    halves = []
    for c in range(2):
        t = jnp.where((idx_t >= c * HALF) & (idx_t < (c + 1) * HALF),
                      idx_t - c * HALF, TRASH)
        halves.append(jnp.pad(t, (0, P - NE * K), constant_values=TRASH))
    scat_idx = jnp.stack(halves)

    node_h = _mlp_rows(nodes2, params["nodeEnc"], norm=True)
    en_h = _mlp_rows(en2, params["elemNodeEnc"], norm=True)
    ex_h = _mlp_rows(ex2, params["elemEnc"], norm=True)

    for lp in params["layers"]:
        gathered = _sc_gather_kernel()(node_h, gat_idx)
        ex_h, interim = _layer_tc(lp, gathered, en_h, ex_h)
        node_h = _sc_scatter_kernel()(interim, scat_idx, node_h)

    out = _mlp_rows(node_h, params["nodeDec"], norm=False)
    return out[None]


# speedup vs baseline: 5.0119x; 1.0967x over previous
import functools

import jax
import jax.numpy as jnp
from jax import lax
from jax.experimental import pallas as pl
from jax.experimental.pallas import tpu as pltpu
from jax.experimental.pallas import tpu_sc as plsc

N = 50000
NE = 100000
K = 4
H = 64

NWORK = 32
CH = 128
PER_W = 12544
P = NWORK * PER_W
NCH_G = PER_W // CH

HALF = N // 2
TRASH = HALF
ACC_ROWS = HALF + 8
PER_T = P // 16
NCH_S = PER_T // CH
ROWS_T = 1562


def _gelu(x):
    return 0.5 * x * (1.0 + lax.erf(x * 0.7071067811865476))


def _ln(y, g, b):
    m = jnp.mean(y, axis=-1, keepdims=True)
    d = y - m
    v = jnp.mean(d * d, axis=-1, keepdims=True)
    return d * lax.rsqrt(v + 1e-5) * g + b


def _dot(a, b):
    return jnp.dot(a, b, preferred_element_type=jnp.float32)


def _mlp_rows(x, p, norm):
    R, nin = x.shape
    nout = p["l3"]["W"].shape[1]
    BR = 4000 if R % 4000 == 0 else 5000
    w = [p["l1"]["W"], p["l1"]["b"].reshape(1, -1),
         p["l2"]["W"], p["l2"]["b"].reshape(1, -1),
         p["l3"]["W"], p["l3"]["b"].reshape(1, -1)]
    if norm:
        w += [p["g"].reshape(1, -1), p["be"].reshape(1, -1)]

    def body(x_ref, *refs):
        o_ref = refs[-1]
        ws = [r[...] for r in refs[:-1]]
        h = _gelu(_dot(x_ref[...], ws[0]) + ws[1])
        h = _gelu(_dot(h, ws[2]) + ws[3])
        y = _dot(h, ws[4]) + ws[5]
        if norm:
            y = _ln(y, ws[6], ws[7])
        o_ref[...] = y

    full = lambda a: pl.BlockSpec(a.shape, lambda i: (0,) * a.ndim)
    return pl.pallas_call(
        body,
        grid=(R // BR,),
        in_specs=[pl.BlockSpec((BR, nin), lambda i: (i, 0))] + [full(a) for a in w],
        out_specs=pl.BlockSpec((BR, nout), lambda i: (i, 0)),
        out_shape=jax.ShapeDtypeStruct((R, nout), jnp.float32),
    )(x, *w)


def _layer_tc(lp, gathered, en_h, ex_h):
    BE = 1000
    BR = BE * K

    def split_w(p):
        W1 = p["l1"]["W"]
        return [W1[:H], W1[H:2 * H], W1[2 * H:], p["l1"]["b"].reshape(1, H),
                p["l2"]["W"], p["l2"]["b"].reshape(1, H),
                p["l3"]["W"], p["l3"]["b"].reshape(1, H),
                p["g"].reshape(1, H), p["be"].reshape(1, H)]

    ew = split_w(lp["elem"])
    nw = split_w(lp["node"])

    def bc4(t):
        return jnp.broadcast_to(t[:, None, :], (BE, K, H)).reshape(BR, H)

    def body(g_ref, en_ref, ex_ref, *refs):
        exo_ref, int_ref = refs[-2], refs[-1]
        ws = [r[...] for r in refs[:-2]]
        e = ws[:10]
        n = ws[10:]
        g = g_ref[...]
        en = en_ref[...]
        ex = ex_ref[...]
        gg_e = _dot(g, e[0]) + _dot(en, e[1])
        pre = gg_e + bc4(_dot(ex, e[2])) + e[3]
        h = _gelu(pre)
        h = _gelu(_dot(h, e[4]) + e[5])
        y = _ln(_dot(h, e[6]) + e[7], e[8], e[9])
        exn = ex + jnp.mean(y.reshape(BE, K, H), axis=1)
        exo_ref[...] = exn
        gg_n = _dot(g, n[0]) + _dot(en, n[1])
        pre2 = gg_n + bc4(_dot(exn, n[2])) + n[3]
        h2 = _gelu(pre2)
        h2 = _gelu(_dot(h2, n[4]) + n[5])
        int_ref[...] = _ln(_dot(h2, n[6]) + n[7], n[8], n[9])

    full = lambda a: pl.BlockSpec(a.shape, lambda i: (0,) * a.ndim)
    return pl.pallas_call(
        body,
        grid=(NE // BE,),
        in_specs=[pl.BlockSpec((BR, H), lambda i: (i, 0)),
                  pl.BlockSpec((BR, H), lambda i: (i, 0)),
                  pl.BlockSpec((BE, H), lambda i: (i, 0))]
                 + [full(a) for a in ew + nw],
        out_specs=[pl.BlockSpec((BE, H), lambda i: (i, 0)),
                   pl.BlockSpec((BR, H), lambda i: (i, 0))],
        out_shape=[jax.ShapeDtypeStruct((NE, H), jnp.float32),
                   jax.ShapeDtypeStruct((P, H), jnp.float32)],
    )(gathered, en_h, ex_h, *ew, *nw)


NB = 4
NBS = 3


@functools.cache
def _sc_gather_kernel():
    mesh = plsc.VectorSubcoreMesh(core_axis_name="c", subcore_axis_name="s")

    @functools.partial(
        pl.kernel,
        out_type=jax.ShapeDtypeStruct((P, H), jnp.float32),
        mesh=mesh,
        scratch_types=[pltpu.VMEM((NCH_G, CH), jnp.int32),
                       pltpu.VMEM((NB, CH, H), jnp.float32),
                       pltpu.SemaphoreType.DMA((NB,)),
                       pltpu.SemaphoreType.DMA((NB,))],
        compiler_params=pltpu.CompilerParams(use_tc_tiling_on_sc=False),
    )
    def _sc_gather(tab, idx2d, out, idx_v, rows_v, gsem, wsem):
        c = lax.axis_index("c")
        s = lax.axis_index("s")
        wid = s * 2 + c
        base = wid * PER_W
        pltpu.sync_copy(idx2d.at[pl.ds(wid * NCH_G, NCH_G)], idx_v)

        def g_desc(i):
            slot = lax.rem(i, NB)
            return pltpu.make_async_copy(tab.at[idx_v.at[i]],
                                         rows_v.at[slot], gsem.at[slot])

        def w_desc(i):
            slot = lax.rem(i, NB)
            return pltpu.make_async_copy(rows_v.at[slot],
                                         out.at[pl.ds(base + i * CH, CH)],
                                         wsem.at[slot])

        g_desc(0).start()
        g_desc(1).start()

        def body(i, carry):
            @pl.when(i + 2 < NCH_G)
            def _():
                @pl.when(i >= 2)
                def _():
                    w_desc(i - 2).wait()

                g_desc(i + 2).start()

            g_desc(i).wait()
            w_desc(i).start()
            return carry

        lax.fori_loop(0, NCH_G, body, 0)
        for k in range(NCH_G - 4, NCH_G):
            w_desc(k).wait()

    return _sc_gather


@functools.cache
def _sc_scatter_kernel():
    mesh = plsc.VectorSubcoreMesh(core_axis_name="c", subcore_axis_name="s")

    @functools.partial(
        pl.kernel,
        out_type=jax.ShapeDtypeStruct((N, H), jnp.float32),
        mesh=mesh,
        scratch_types=[pltpu.VMEM((NBS, CH), jnp.int32),
                       pltpu.VMEM((NBS, CH, H), jnp.float32),
                       pltpu.VMEM_SHARED((ACC_ROWS, H), jnp.float32),
                       pltpu.SemaphoreType.DMA((NBS,)),
                       pltpu.SemaphoreType.DMA((NBS,)),
                       pltpu.SemaphoreType.DMA((NBS,))],
        compiler_params=pltpu.CompilerParams(use_tc_tiling_on_sc=False),
    )
    def _sc_scatter(interim, scidx, nodes, out, idx_v, rows_v, acc,
                    isem, lsem, ssem):
        c = lax.axis_index("c")
        s = lax.axis_index("s")
        pltpu.sync_copy(nodes.at[pl.ds(c * HALF + s * ROWS_T, ROWS_T)],
                        acc.at[pl.ds(s * ROWS_T, ROWS_T)])

        @pl.when(s == 15)
        def _():
            pltpu.sync_copy(
                nodes.at[pl.ds(c * HALF + 16 * ROWS_T, HALF - 16 * ROWS_T)],
                acc.at[pl.ds(16 * ROWS_T, HALF - 16 * ROWS_T)])

        plsc.subcore_barrier()

        def i_desc(i):
            slot = lax.rem(i, NBS)
            return pltpu.make_async_copy(scidx.at[c, s * NCH_S + i],
                                         idx_v.at[slot], isem.at[slot])

        def l_desc(i):
            slot = lax.rem(i, NBS)
            return pltpu.make_async_copy(
                interim.at[pl.ds(s * PER_T + i * CH, CH)],
                rows_v.at[slot], lsem.at[slot])

        def s_desc(i):
            slot = lax.rem(i, NBS)
            return pltpu.make_async_copy(rows_v.at[slot],
                                         acc.at[idx_v.at[slot]], ssem.at[slot])

        def fire(i):
            i_desc(i).start()
            l_desc(i).start()

        fire(0)
        fire(1)

        def body(i, carry):
            @pl.when(i + 2 < NCH_S)
            def _():
                @pl.when(i >= 1)
                def _():
                    s_desc(i - 1).wait()

                fire(i + 2)

            i_desc(i).wait()
            l_desc(i).wait()
            s_desc(i).start(add=True)
            return carry

        lax.fori_loop(0, NCH_S, body, 0)
        for k in range(NCH_S - 3, NCH_S):
            s_desc(k).wait()
        plsc.subcore_barrier()
        pltpu.sync_copy(acc.at[pl.ds(s * ROWS_T, ROWS_T)],
                        out.at[pl.ds(c * HALF + s * ROWS_T, ROWS_T)])

        @pl.when(s == 15)
        def _():
            pltpu.sync_copy(
                acc.at[pl.ds(16 * ROWS_T, HALF - 16 * ROWS_T)],
                out.at[pl.ds(c * HALF + 16 * ROWS_T, HALF - 16 * ROWS_T)])

    return _sc_scatter


def kernel(nodes, elemNodes, elems, elemConn, params):
    nodes2 = nodes[0]
    en2 = elemNodes.reshape(NE * K, -1)
    ex2 = elems[0]
    conn = elemConn[0]

    gat_idx = jnp.pad(conn.reshape(-1), (0, P - NE * K)).reshape(P // CH, CH)
    idx_t = conn.T.reshape(-1)
    halves = []
    for c in range(2):
        t = jnp.where((idx_t >= c * HALF) & (idx_t < (c + 1) * HALF),
                      idx_t - c * HALF, TRASH)
        halves.append(jnp.pad(t, (0, P - NE * K), constant_values=TRASH))
    scat_idx = jnp.stack(halves).reshape(2, P // CH, CH)

    node_h = _mlp_rows(nodes2, params["nodeEnc"], norm=True)
    en_h = _mlp_rows(en2, params["elemNodeEnc"], norm=True)
    ex_h = _mlp_rows(ex2, params["elemEnc"], norm=True)

    for lp in params["layers"]:
        gathered = _sc_gather_kernel()(node_h, gat_idx)
        ex_h, interim = _layer_tc(lp, gathered, en_h, ex_h)
        node_h = _sc_scatter_kernel()(interim, scat_idx, node_h)

    out = _mlp_rows(node_h, params["nodeDec"], norm=False)
    return out[None]


# speedup vs baseline: 9.2005x; 1.8357x over previous
import functools

import jax
import jax.numpy as jnp
from jax import lax
from jax.experimental import pallas as pl
from jax.experimental.pallas import tpu as pltpu
from jax.experimental.pallas import tpu_sc as plsc

N = 50000
NE = 100000
K = 4
H = 64

NWORK = 32
CH = 128
PER_W = 12544
P = NWORK * PER_W
NCH_G = PER_W // CH

NEP2 = P // K
HALF = N // 2
TRASH = HALF
ACC_ROWS = HALF + 8
PER_T = P // 16
NCH_S = PER_T // CH
ROWS_T = 1562


def _gelu(x):
    return 0.5 * x * (1.0 + lax.erf(x * 0.7071067811865476))


def _dot(a, b):
    return jnp.dot(a, b, preferred_element_type=jnp.float32)


def _bd(w):
    return jnp.kron(jnp.eye(2, dtype=jnp.float32), w)


def _jp():
    return jnp.kron(jnp.eye(2, dtype=jnp.float32),
                    jnp.full((H, H), 1.0 / H, jnp.float32))


def _t2(v):
    return jnp.concatenate([v, v]).reshape(1, -1)


def _ln_p(y, jp, g, b):
    mu = _dot(y, jp)
    d = y - mu
    var = _dot(d * d, jp)
    return d * lax.rsqrt(var + 1e-5) * g + b


def _mlp_rows(x, p, norm):
    R, nin = x.shape
    nout = p["l3"]["W"].shape[1]
    BR = 4000 if R % 4000 == 0 else (5000 if R % 5000 == 0 else 2500)
    w = [_bd(p["l1"]["W"]), _t2(p["l1"]["b"]),
         _bd(p["l2"]["W"]), _t2(p["l2"]["b"]),
         _bd(p["l3"]["W"]), _t2(p["l3"]["b"]), _jp()]
    if norm:
        w += [_t2(p["g"]), _t2(p["be"])]

    def body(x_ref, *refs):
        o_ref = refs[-1]
        ws = [r[...] for r in refs[:-1]]
        h = _gelu(_dot(x_ref[...], ws[0]) + ws[1])
        h = _gelu(_dot(h, ws[2]) + ws[3])
        y = _dot(h, ws[4]) + ws[5]
        if norm:
            y = _ln_p(y, ws[6], ws[7], ws[8])
        o_ref[...] = y

    full = lambda a: pl.BlockSpec(a.shape, lambda i: (0,) * a.ndim)
    return pl.pallas_call(
        body,
        grid=(R // BR,),
        in_specs=[pl.BlockSpec((BR, nin), lambda i: (i, 0))] + [full(a) for a in w],
        out_specs=pl.BlockSpec((BR, 2 * nout), lambda i: (i, 0)),
        out_shape=jax.ShapeDtypeStruct((R, 2 * nout), jnp.float32),
    )(x, *w)


def _layer_tc(lp, g3, en3, ex_p):
    BX = 1000
    BR = K * BX

    def split_w(p):
        W1 = p["l1"]["W"]
        return [_bd(W1[:H]), _bd(W1[H:2 * H]), _bd(W1[2 * H:]),
                _t2(p["l1"]["b"]), _bd(p["l2"]["W"]), _t2(p["l2"]["b"]),
                _bd(p["l3"]["W"]), _t2(p["l3"]["b"]), _t2(p["g"]), _t2(p["be"])]

    ew = split_w(lp["elem"])
    nw = split_w(lp["node"])
    jp = _jp()

    def bc4(tp):
        return jnp.broadcast_to(tp[None], (K, BX, 2 * H)).reshape(BR, 2 * H)

    def body(g_ref, en_ref, ex_ref, jp_ref, *refs):
        exo_ref, int_ref = refs[-2], refs[-1]
        ws = [r[...] for r in refs[:-2]]
        e = ws[:10]
        n = ws[10:]
        g = g_ref[...].reshape(BR, 2 * H)
        en = en_ref[...].reshape(BR, 2 * H)
        ex = ex_ref[...]
        jpm = jp_ref[...]
        pre = _dot(g, e[0]) + _dot(en, e[1]) + bc4(_dot(ex, e[2])) + e[3]
        h = _gelu(pre)
        h = _gelu(_dot(h, e[4]) + e[5])
        y = _ln_p(_dot(h, e[6]) + e[7], jpm, e[8], e[9])
        y3 = y.reshape(K, BX, 2 * H)
        exn = ex + (y3[0] + y3[1] + y3[2] + y3[3]) * 0.25
        exo_ref[...] = exn
        pre2 = _dot(g, n[0]) + _dot(en, n[1]) + bc4(_dot(exn, n[2])) + n[3]
        h2 = _gelu(pre2)
        h2 = _gelu(_dot(h2, n[4]) + n[5])
        y2 = _ln_p(_dot(h2, n[6]) + n[7], jpm, n[8], n[9])
        int_ref[...] = y2.reshape(K, BX, 2 * H)

    full = lambda a: pl.BlockSpec(a.shape, lambda i: (0,) * a.ndim)
    return pl.pallas_call(
        body,
        grid=(NE // (2 * BX),),
        in_specs=[pl.BlockSpec((K, BX, 2 * H), lambda i: (0, i, 0)),
                  pl.BlockSpec((K, BX, 2 * H), lambda i: (0, i, 0)),
                  pl.BlockSpec((BX, 2 * H), lambda i: (i, 0)),
                  full(jp)]
                 + [full(a) for a in ew + nw],
        out_specs=[pl.BlockSpec((BX, 2 * H), lambda i: (i, 0)),
                   pl.BlockSpec((K, BX, 2 * H), lambda i: (0, i, 0))],
        out_shape=[jax.ShapeDtypeStruct((NE // 2, 2 * H), jnp.float32),
                   jax.ShapeDtypeStruct((K, NEP2 // 2, 2 * H), jnp.float32)],
    )(g3, en3, ex_p, jp, *ew, *nw)


NB = 6
NBS = 3


@functools.cache
def _sc_gather_kernel():
    mesh = plsc.VectorSubcoreMesh(core_axis_name="c", subcore_axis_name="s")

    @functools.partial(
        pl.kernel,
        out_type=jax.ShapeDtypeStruct((P, H), jnp.float32),
        mesh=mesh,
        scratch_types=[pltpu.VMEM((NCH_G, CH), jnp.int32),
                       pltpu.VMEM((NB, CH, H), jnp.float32),
                       pltpu.SemaphoreType.DMA((NB,)),
                       pltpu.SemaphoreType.DMA((NB,))],
        compiler_params=pltpu.CompilerParams(use_tc_tiling_on_sc=False),
    )
    def _sc_gather(tab, idx2d, out, idx_v, rows_v, gsem, wsem):
        c = lax.axis_index("c")
        s = lax.axis_index("s")
        wid = s * 2 + c
        base = wid * PER_W
        pltpu.sync_copy(idx2d.at[pl.ds(wid * NCH_G, NCH_G)], idx_v)

        def g_desc(i):
            slot = lax.rem(i, NB)
            return pltpu.make_async_copy(tab.at[idx_v.at[i]],
                                         rows_v.at[slot], gsem.at[slot])

        def w_desc(i):
            slot = lax.rem(i, NB)
            return pltpu.make_async_copy(rows_v.at[slot],
                                         out.at[pl.ds(base + i * CH, CH)],
                                         wsem.at[slot])

        for j in range(NB - 2):
            g_desc(j).start()

        def body(i, carry):
            nxt = i + (NB - 2)

            @pl.when(nxt < NCH_G)
            def _():
                @pl.when(nxt >= NB)
                def _():
                    w_desc(nxt - NB).wait()

                g_desc(nxt).start()

            g_desc(i).wait()
            w_desc(i).start()
            return carry

        lax.fori_loop(0, NCH_G, body, 0)
        for k in range(NCH_G - NB, NCH_G):
            w_desc(k).wait()

    return _sc_gather


@functools.cache
def _sc_scatter_kernel():
    mesh = plsc.VectorSubcoreMesh(core_axis_name="c", subcore_axis_name="s")

    @functools.partial(
        pl.kernel,
        out_type=jax.ShapeDtypeStruct((N, H), jnp.float32),
        mesh=mesh,
        scratch_types=[pltpu.VMEM((NBS, CH), jnp.int32),
                       pltpu.VMEM((NBS, CH, H), jnp.float32),
                       pltpu.VMEM_SHARED((ACC_ROWS, H), jnp.float32),
                       pltpu.SemaphoreType.DMA((NBS,)),
                       pltpu.SemaphoreType.DMA((NBS,)),
                       pltpu.SemaphoreType.DMA((NBS,))],
        compiler_params=pltpu.CompilerParams(use_tc_tiling_on_sc=False),
    )
    def _sc_scatter(interim, scidx, nodes, out, idx_v, rows_v, acc,
                    isem, lsem, ssem):
        c = lax.axis_index("c")
        s = lax.axis_index("s")
        pltpu.sync_copy(nodes.at[pl.ds(c * HALF + s * ROWS_T, ROWS_T)],
                        acc.at[pl.ds(s * ROWS_T, ROWS_T)])

        @pl.when(s == 15)
        def _():
            pltpu.sync_copy(
                nodes.at[pl.ds(c * HALF + 16 * ROWS_T, HALF - 16 * ROWS_T)],
                acc.at[pl.ds(16 * ROWS_T, HALF - 16 * ROWS_T)])

        plsc.subcore_barrier()

        def i_desc(i):
            slot = lax.rem(i, NBS)
            return pltpu.make_async_copy(scidx.at[c, s * NCH_S + i],
                                         idx_v.at[slot], isem.at[slot])

        def l_desc(i):
            slot = lax.rem(i, NBS)
            return pltpu.make_async_copy(
                interim.at[pl.ds(s * PER_T + i * CH, CH)],
                rows_v.at[slot], lsem.at[slot])

        def s_desc(i):
            slot = lax.rem(i, NBS)
            return pltpu.make_async_copy(rows_v.at[slot],
                                         acc.at[idx_v.at[slot]], ssem.at[slot])

        def fire(i):
            i_desc(i).start()
            l_desc(i).start()

        fire(0)
        fire(1)

        def body(i, carry):
            @pl.when(i + 2 < NCH_S)
            def _():
                @pl.when(i >= 1)
                def _():
                    s_desc(i - 1).wait()

                fire(i + 2)

            i_desc(i).wait()
            l_desc(i).wait()
            s_desc(i).start(add=True)
            return carry

        lax.fori_loop(0, NCH_S, body, 0)
        for k in range(NCH_S - 3, NCH_S):
            s_desc(k).wait()
        plsc.subcore_barrier()
        pltpu.sync_copy(acc.at[pl.ds(s * ROWS_T, ROWS_T)],
                        out.at[pl.ds(c * HALF + s * ROWS_T, ROWS_T)])

        @pl.when(s == 15)
        def _():
            pltpu.sync_copy(
                acc.at[pl.ds(16 * ROWS_T, HALF - 16 * ROWS_T)],
                out.at[pl.ds(c * HALF + 16 * ROWS_T, HALF - 16 * ROWS_T)])

    return _sc_scatter


def kernel(nodes, elemNodes, elems, elemConn, params):
    nodes2 = nodes[0]
    ex2 = elems[0]
    conn = elemConn[0]
    en_km = jnp.transpose(elemNodes[0], (1, 0, 2)).reshape(NE * K // 2, 32)

    gat_idx = jnp.pad(conn.T, ((0, 0), (0, NEP2 - NE))).reshape(P // CH, CH)

    j = jnp.arange(NE)[None, :] * K + jnp.arange(K)[:, None]
    tgt = conn[j % NE, j // NE]
    tgt = jnp.pad(tgt, ((0, 0), (0, NEP2 - NE)),
                  constant_values=N).reshape(-1)
    halves = []
    for c in range(2):
        halves.append(jnp.where((tgt >= c * HALF) & (tgt < (c + 1) * HALF),
                                tgt - c * HALF, TRASH))
    scat_idx = jnp.stack(halves).reshape(2, P // CH, CH)

    node_p = _mlp_rows(nodes2.reshape(N // 2, 256),
                       params["nodeEnc"], norm=True)
    en_p = _mlp_rows(en_km, params["elemNodeEnc"], norm=True)
    ex_p = _mlp_rows(ex2.reshape(NE // 2, 32),
                     params["elemEnc"], norm=True)

    en3 = en_p.reshape(K, NE // 2, 2 * H)
    node_h = node_p.reshape(N, H)
    for lp in params["layers"]:
        gathered = _sc_gather_kernel()(node_h, gat_idx)
        ex_p, interim3 = _layer_tc(lp, gathered.reshape(K, NEP2 // 2, 2 * H),
                                   en3, ex_p)
        node_h = _sc_scatter_kernel()(interim3.reshape(P, H),
                                      scat_idx, node_h)

    out = _mlp_rows(node_h.reshape(N // 2, 2 * H),
                    params["nodeDec"], norm=False)
    return out.reshape(1, N, 128)
